# baseline (speedup 1.0000x reference)
import jax
import jax.numpy as jnp
from jax.experimental import pallas as pl
from jax.experimental.pallas import tpu as pltpu

_ROWS = 40320
_FEAT = 100
_BLOCK_ROWS = 2520


def _mask_fill_body(mask_ref, x_ref, o_ref):
    m = mask_ref[...]
    x = x_ref[...]
    o_ref[...] = jnp.where(m[None, None, :, :], x, jnp.float32(0.0))


def kernel(x, mask, dim):
    del dim
    b0, b1, rows, feat = x.shape
    mask2d = mask.reshape(rows, 1)
    grid = (b0 * b1, rows // _BLOCK_ROWS)
    out = pl.pallas_call(
        _mask_fill_body,
        grid=grid,
        in_specs=[
            pl.BlockSpec((_BLOCK_ROWS, 1), lambda i, j: (j, 0)),
            pl.BlockSpec(
                (1, 1, _BLOCK_ROWS, feat),
                lambda i, j: (i // b1, i % b1, j, 0),
            ),
        ],
        out_specs=pl.BlockSpec(
            (1, 1, _BLOCK_ROWS, feat),
            lambda i, j: (i // b1, i % b1, j, 0),
        ),
        out_shape=jax.ShapeDtypeStruct(x.shape, x.dtype),
    )(mask2d, x)
    return out
